# Initial kernel scaffold; baseline (speedup 1.0000x reference)
import jax
import jax.numpy as jnp
from jax.experimental import pallas as pl


def kernel(x, edge_attr, edge_sbf, node_rbf, edge_index, batch, edge_index_0, atom_batch, params):
    raise NotImplementedError("write your pallas kernel here")


# speedup vs baseline: 1.5472x; 1.5472x over previous
import functools

import jax
import jax.numpy as jnp
import numpy as np
from jax.experimental import pallas as pl

N = 10000
E = 160000
F = 128
EMB = 128
RBF = 16
SBF = 128
H = 8
DH = 16
L = 2
G = 64

BE = 2000
NEB = E // BE
BN = 2000
NNB = N // BN


def _silu(x):
    return x * jax.nn.sigmoid(x)


def _edge_pre_body(attr, sbf, w0, b0, w1, b1, we0, be0, we1, be1, ws0, ws1,
                   ee0_o, ee1_o, sl0_o, sl1_o):
    h = _silu(attr[...] @ w0[...] + b0[...])
    ea = h @ w1[...] + b1[...]
    ee0_o[...] = ea @ we0[...] + be0[...]
    ee1_o[...] = ea @ we1[...] + be1[...]
    sl0_o[...] = sbf[...] @ ws0[...]
    sl1_o[...] = sbf[...] @ ws1[...]


def _edge_pre(edge_attr, edge_sbf, p):
    row = lambda i: (i, 0)
    full = lambda i: (0, 0)
    eb = pl.BlockSpec((BE, F), row)
    wb = pl.BlockSpec((F, F), full)
    bb = pl.BlockSpec((1, F), full)
    sb = pl.BlockSpec((F, H), full)
    ob8 = pl.BlockSpec((BE, H), row)
    return pl.pallas_call(
        _edge_pre_body,
        grid=(NEB,),
        in_specs=[eb, eb, wb, bb, wb, bb, wb, bb, wb, bb, sb, sb],
        out_specs=[eb, eb, ob8, ob8],
        out_shape=[
            jax.ShapeDtypeStruct((E, F), jnp.float32),
            jax.ShapeDtypeStruct((E, F), jnp.float32),
            jax.ShapeDtypeStruct((E, H), jnp.float32),
            jax.ShapeDtypeStruct((E, H), jnp.float32),
        ],
    )(edge_attr, edge_sbf,
      p['edgenn'][0]['W'], p['edgenn'][0]['b'][None, :],
      p['edgenn'][1]['W'], p['edgenn'][1]['b'][None, :],
      p['convs'][0]['e']['W'], p['convs'][0]['e']['b'][None, :],
      p['convs'][1]['e']['W'], p['convs'][1]['e']['b'][None, :],
      p['convs'][0]['sbf'], p['convs'][1]['sbf'])


def _qkv_body(x, wq, bq, wk, bk, wv, bv, q_o, k_o, v_o):
    xv = x[...]
    q_o[...] = xv @ wq[...] + bq[...]
    k_o[...] = xv @ wk[...] + bk[...]
    v_o[...] = xv @ wv[...] + bv[...]


def _qkv(x, cp):
    row = lambda i: (i, 0)
    full = lambda i: (0, 0)
    nb = pl.BlockSpec((BN, F), row)
    wb = pl.BlockSpec((F, F), full)
    bb = pl.BlockSpec((1, F), full)
    return pl.pallas_call(
        _qkv_body,
        grid=(NNB,),
        in_specs=[nb, wb, bb, wb, bb, wb, bb],
        out_specs=[nb, nb, nb],
        out_shape=[jax.ShapeDtypeStruct((N, F), jnp.float32)] * 3,
    )(x, cp['q']['W'], cp['q']['b'][None, :],
      cp['k']['W'], cp['k']['b'][None, :],
      cp['v']['W'], cp['v']['b'][None, :])


def _edge_msg_body(qd, ks, vs, ee, sl, mred, mexp, num_o, den_o):
    eev = ee[...]
    k = ks[...] + eev
    v = vs[...] + eev
    logit = (qd[...] * k) @ mred[...] * (1.0 / np.sqrt(float(DH))) + sl[...]
    w = jnp.exp(logit)
    den_o[...] = w
    num_o[...] = (w @ mexp[...]) * v


def _edge_msg(qd, ks, vs, ee, sl, mred, mexp):
    row = lambda i: (i, 0)
    full = lambda i: (0, 0)
    eb = pl.BlockSpec((BE, F), row)
    hb = pl.BlockSpec((BE, H), row)
    return pl.pallas_call(
        _edge_msg_body,
        grid=(NEB,),
        in_specs=[eb, eb, eb, eb, hb,
                  pl.BlockSpec((F, H), full), pl.BlockSpec((H, F), full)],
        out_specs=[eb, hb],
        out_shape=[
            jax.ShapeDtypeStruct((E, F), jnp.float32),
            jax.ShapeDtypeStruct((E, H), jnp.float32),
        ],
    )(qd, ks, vs, ee, sl, mred, mexp)


def _node_phase_body(num, den, res0, rbf, P, mexp, wr, wo, bo,
                     wbf0, bbf0, wbf1, bbf1, wd, bd,
                     wa00, ba00, wa01, ba01, wa10, ba10, wa11, ba11,
                     out_o):
    agg = num[...] / (den[...] @ mexp[...] + 1e-16)
    gate = rbf[...] @ wr[...]
    t = (agg * gate) @ wo[...] + bo[...]

    Pv = P[...]
    cnt = jnp.sum(Pv, axis=0)
    denom = jnp.maximum(cnt, 1.0) * float(F)
    s_g = jax.lax.dot_general(Pv, t, (((0,), (0,)), ((), ())))
    mean_g = jnp.sum(s_g, axis=1) / denom
    mean_n = Pv @ mean_g[:, None]
    xc = t - mean_n
    r = jnp.sum(xc * xc, axis=1, keepdims=True)
    v_g = jax.lax.dot_general(Pv, r, (((0,), (0,)), ((), ())))
    var_g = v_g / denom[:, None]
    var_n = Pv @ var_g
    t = xc / jnp.sqrt(var_n + 1e-8)

    h = _silu(t @ wbf0[...] + bbf0[...])
    h = _silu(h @ wbf1[...] + bbf1[...])
    t = t + h
    t = _silu(t @ wd[...] + bd[...])
    t = t + res0[...]
    h = _silu(t @ wa00[...] + ba00[...])
    h = _silu(h @ wa01[...] + ba01[...])
    t = t + h
    h = _silu(t @ wa10[...] + ba10[...])
    h = _silu(h @ wa11[...] + ba11[...])
    out_o[...] = t + h


def _node_phase(num, den, res0, node_rbf, P, mexp, p, li):
    cp = p['convs'][li]
    bf = p['bf_skip'][li]
    af = p['af_skip'][li]
    full = lambda: pl.BlockSpec(memory_space=pl.ANY) if False else None
    def b(shape):
        return pl.BlockSpec(shape, lambda: (0,) * len(shape))
    args = [
        num, den, res0, node_rbf, P, mexp,
        cp['rbf'], cp['o']['W'], cp['o']['b'][None, :],
        bf[0]['W'], bf[0]['b'][None, :], bf[1]['W'], bf[1]['b'][None, :],
        p['dense_bf'][li]['W'], p['dense_bf'][li]['b'][None, :],
        af[0][0]['W'], af[0][0]['b'][None, :], af[0][1]['W'], af[0][1]['b'][None, :],
        af[1][0]['W'], af[1][0]['b'][None, :], af[1][1]['W'], af[1][1]['b'][None, :],
    ]
    in_specs = [b(tuple(a.shape)) for a in args]
    return pl.pallas_call(
        _node_phase_body,
        in_specs=in_specs,
        out_specs=b((N, F)),
        out_shape=jax.ShapeDtypeStruct((N, F), jnp.float32),
    )(*args)


def _readout_pre_body(x, rbf, wu, bu, wg, y_o):
    up = _silu(x[...] @ wu[...] + bu[...])
    gate = _silu(rbf[...] @ wg[...])
    y_o[...] = up * gate


def _readout_pre(x, node_rbf, p):
    row = lambda i: (i, 0)
    full = lambda i: (0, 0)
    return pl.pallas_call(
        _readout_pre_body,
        grid=(NNB,),
        in_specs=[pl.BlockSpec((BN, F), row), pl.BlockSpec((BN, RBF), row),
                  pl.BlockSpec((F, 2 * F), full), pl.BlockSpec((1, 2 * F), full),
                  pl.BlockSpec((RBF, 2 * F), full)],
        out_specs=pl.BlockSpec((BN, 2 * F), row),
        out_shape=jax.ShapeDtypeStruct((N, 2 * F), jnp.float32),
    )(x, node_rbf, p['up']['W'], p['up']['b'][None, :],
      p['readout']['gate'])


def _readout_post_body(h, P0, w0, b0, w1, b1, w2, b2, wo, bo, out_o):
    t = h[...]
    t = _silu(t @ w0[...] + b0[...])
    t = _silu(t @ w1[...] + b1[...])
    t = _silu(t @ w2[...] + b2[...])
    Pv = P0[...]
    cnt = jnp.sum(Pv, axis=0)
    pooled = jax.lax.dot_general(Pv, t, (((0,), (0,)), ((), ())))
    pooled = pooled / jnp.maximum(cnt, 1.0)[:, None]
    out_o[...] = pooled @ wo[...] + bo[...]


def _readout_post(h, P0, p):
    rp = p['readout']
    def b(shape):
        return pl.BlockSpec(shape, lambda: (0,) * len(shape))
    args = [h, P0,
            rp['mlp'][0]['W'], rp['mlp'][0]['b'][None, :],
            rp['mlp'][1]['W'], rp['mlp'][1]['b'][None, :],
            rp['mlp'][2]['W'], rp['mlp'][2]['b'][None, :],
            rp['out']['W'], rp['out']['b'][None, :]]
    return pl.pallas_call(
        _readout_post_body,
        in_specs=[b(tuple(a.shape)) for a in args],
        out_specs=b((G, 1)),
        out_shape=jax.ShapeDtypeStruct((G, 1), jnp.float32),
    )(*args)


def kernel(x, edge_attr, edge_sbf, node_rbf, edge_index, batch, edge_index_0,
           atom_batch, params):
    p = params
    src = edge_index[0]
    dst = edge_index[1]
    src0 = edge_index_0[0]
    dst0 = edge_index_0[1]

    mred = np.zeros((F, H), np.float32)
    for h in range(H):
        mred[h * DH:(h + 1) * DH, h] = 1.0
    mred = jnp.asarray(mred)
    mexp = jnp.asarray(mred.T)

    P = jax.nn.one_hot(batch, G, dtype=jnp.float32)
    P0 = jax.nn.one_hot(atom_batch, G, dtype=jnp.float32)

    ee0, ee1, sl0, sl1 = _edge_pre(edge_attr, edge_sbf, p)
    ees = [ee0, ee1]
    sls = [sl0, sl1]

    out = x
    for li in range(L):
        q, k, v = _qkv(out, p['convs'][li])
        qd = q[dst]
        ks = k[src]
        vs = v[src]
        num, den = _edge_msg(qd, ks, vs, ees[li], sls[li], mred, mexp)
        num_n = jax.ops.segment_sum(num, dst, N)
        den_n = jax.ops.segment_sum(den, dst, N)
        out = _node_phase(num_n, den_n, out, node_rbf, P, mexp, p, li)

    y = _readout_pre(out, node_rbf, p)
    h = jax.ops.segment_sum(y[src0], dst0, N)
    res = _readout_post(h, P0, p)
    return res.reshape(-1)


# speedup vs baseline: 5.1848x; 3.3511x over previous
import functools

import jax
import jax.numpy as jnp
import numpy as np
from jax import lax
from jax.experimental import pallas as pl
from jax.experimental.pallas import tpu as pltpu
from jax.experimental.pallas import tpu_sc as plsc

N = 10000
E = 160000
F = 128
EMB = 128
RBF = 16
SBF = 128
H = 8
DH = 16
L = 2
G = 64

BE = 2000
NEB = E // BE
BN = 2000
NNB = N // BN

NC = 2
NS = 16
NW = NC * NS
NPAD = 10240
NZT = NPAD // NS

ND = F + 2 * H

CEW = E // NW
CCH = 200
CNCH = CEW // CCH

REW = E // NS
RCH = 200
RNCH = REW // RCH


def _silu(x):
    return x * jax.nn.sigmoid(x)


def _edge_pre_body(attr, sbf, w0, b0, w1, b1, we0, be0, we1, be1, ws0, ws1,
                   ee0_o, ee1_o, sl0_o, sl1_o):
    h = _silu(attr[...] @ w0[...] + b0[...])
    ea = h @ w1[...] + b1[...]
    ee0_o[...] = ea @ we0[...] + be0[...]
    ee1_o[...] = ea @ we1[...] + be1[...]
    sl0_o[...] = sbf[...] @ ws0[...]
    sl1_o[...] = sbf[...] @ ws1[...]


def _edge_pre(edge_attr, edge_sbf, p):
    row = lambda i: (i, 0)
    full = lambda i: (0, 0)
    eb = pl.BlockSpec((BE, F), row)
    wb = pl.BlockSpec((F, F), full)
    bb = pl.BlockSpec((1, F), full)
    sb = pl.BlockSpec((F, H), full)
    ob8 = pl.BlockSpec((BE, H), row)
    return pl.pallas_call(
        _edge_pre_body,
        grid=(NEB,),
        in_specs=[eb, eb, wb, bb, wb, bb, wb, bb, wb, bb, sb, sb],
        out_specs=[eb, eb, ob8, ob8],
        out_shape=[
            jax.ShapeDtypeStruct((E, F), jnp.float32),
            jax.ShapeDtypeStruct((E, F), jnp.float32),
            jax.ShapeDtypeStruct((E, H), jnp.float32),
            jax.ShapeDtypeStruct((E, H), jnp.float32),
        ],
    )(edge_attr, edge_sbf,
      p['edgenn'][0]['W'], p['edgenn'][0]['b'][None, :],
      p['edgenn'][1]['W'], p['edgenn'][1]['b'][None, :],
      p['convs'][0]['e']['W'], p['convs'][0]['e']['b'][None, :],
      p['convs'][1]['e']['W'], p['convs'][1]['e']['b'][None, :],
      p['convs'][0]['sbf'], p['convs'][1]['sbf'])


def _qkv_body(x, wq, bq, wk, bk, wv, bv, q_o, kv_o):
    xv = x[...]
    q_o[...] = xv @ wq[...] + bq[...]
    k = xv @ wk[...] + bk[...]
    v = xv @ wv[...] + bv[...]
    kv_o[...] = jnp.concatenate([k, v], axis=1)


def _qkv(x, cp):
    row = lambda i: (i, 0)
    full = lambda i: (0, 0)
    nb = pl.BlockSpec((BN, F), row)
    wb = pl.BlockSpec((F, F), full)
    bb = pl.BlockSpec((1, F), full)
    return pl.pallas_call(
        _qkv_body,
        grid=(NNB,),
        in_specs=[nb, wb, bb, wb, bb, wb, bb],
        out_specs=[nb, pl.BlockSpec((BN, 2 * F), row)],
        out_shape=[jax.ShapeDtypeStruct((N, F), jnp.float32),
                   jax.ShapeDtypeStruct((N, 2 * F), jnp.float32)],
    )(x, cp['q']['W'], cp['q']['b'][None, :],
      cp['k']['W'], cp['k']['b'][None, :],
      cp['v']['W'], cp['v']['b'][None, :])


def _sc_convgather_body(q_hbm, kv_hbm, src_r, dst_r, qd_o, kvs_o,
                        srcbuf, dstbuf, qbuf, kvbuf, sem, sem2):
    c = lax.axis_index("c")
    s = lax.axis_index("s")
    w = c * NS + s
    pltpu.sync_copy(src_r.at[w], srcbuf)
    pltpu.sync_copy(dst_r.at[w], dstbuf)

    def body(j, carry):
        base = w * CEW + j * CCH
        cp1 = pltpu.async_copy(q_hbm.at[dstbuf.at[pl.ds(j * CCH, CCH)]], qbuf, sem)
        cp2 = pltpu.async_copy(kv_hbm.at[srcbuf.at[pl.ds(j * CCH, CCH)]], kvbuf, sem2)
        cp1.wait()
        cp2.wait()
        pltpu.sync_copy(qbuf, qd_o.at[pl.ds(base, CCH)])
        pltpu.sync_copy(kvbuf, kvs_o.at[pl.ds(base, CCH)])
        return carry

    lax.fori_loop(0, CNCH, body, 0)


def _sc_convgather(q, kv, src_r, dst_r):
    mesh = plsc.VectorSubcoreMesh(core_axis_name="c", subcore_axis_name="s")
    return pl.kernel(
        _sc_convgather_body,
        out_type=[jax.ShapeDtypeStruct((E, F), jnp.float32),
                  jax.ShapeDtypeStruct((E, 2 * F), jnp.float32)],
        mesh=mesh,
        scratch_types=[
            pltpu.VMEM((CEW,), jnp.int32),
            pltpu.VMEM((CEW,), jnp.int32),
            pltpu.VMEM((CCH, F), jnp.float32),
            pltpu.VMEM((CCH, 2 * F), jnp.float32),
            pltpu.SemaphoreType.DMA,
            pltpu.SemaphoreType.DMA,
        ],
    )(q, kv, src_r, dst_r)


def _edge_msg_body(qd, kvs, ee, sl, mred, mexp, num_o, wex_o):
    eev = ee[...]
    kvv = kvs[...]
    k = kvv[:, :F] + eev
    v = kvv[:, F:] + eev
    logit = (qd[...] * k) @ mred[...] * (1.0 / np.sqrt(float(DH))) + sl[...]
    wex = jnp.exp(logit) @ mexp[...]
    wex_o[...] = wex
    num_o[...] = wex * v


def _edge_msg(qd, kvs, ee, sl, mred, mexp):
    row = lambda i: (i, 0)
    full = lambda i: (0, 0)
    return pl.pallas_call(
        _edge_msg_body,
        grid=(NEB,),
        in_specs=[pl.BlockSpec((BE, F), row), pl.BlockSpec((BE, 2 * F), row),
                  pl.BlockSpec((BE, F), row), pl.BlockSpec((BE, H), row),
                  pl.BlockSpec((F, H), full), pl.BlockSpec((H, F), full)],
        out_specs=[pl.BlockSpec((BE, F), row), pl.BlockSpec((BE, F), row)],
        out_shape=[jax.ShapeDtypeStruct((E, F), jnp.float32),
                   jax.ShapeDtypeStruct((E, F), jnp.float32)],
    )(qd, kvs, ee, sl, mred, mexp)


def _sc_convscatter_body(num_hbm, wex_hbm, dst_r, zeros, num_o, wex_o,
                         acc, dstbuf, rowbuf):
    c = lax.axis_index("c")
    s = lax.axis_index("s")
    pltpu.sync_copy(dst_r.at[s], dstbuf)
    pltpu.sync_copy(zeros.at[pl.ds(s * NZT, NZT)], acc.at[pl.ds(s * NZT, NZT)])
    plsc.subcore_barrier()

    def chunk(j, src_hbm):
        base = s * REW + j * RCH
        pltpu.sync_copy(src_hbm.at[pl.ds(base, RCH)], rowbuf)
        pltpu.sync_copy(rowbuf, acc.at[dstbuf.at[pl.ds(j * RCH, RCH)]],
                        add=True)

    @pl.when(c == 0)
    def _():
        def body(j, carry):
            chunk(j, num_hbm)
            return carry
        lax.fori_loop(0, RNCH, body, 0)

    @pl.when(c == 1)
    def _():
        def body(j, carry):
            chunk(j, wex_hbm)
            return carry
        lax.fori_loop(0, RNCH, body, 0)

    plsc.subcore_barrier()

    @pl.when(c == 0)
    def _():
        pltpu.sync_copy(acc.at[pl.ds(s * NZT, NZT)],
                        num_o.at[pl.ds(s * NZT, NZT)])

    @pl.when(c == 1)
    def _():
        pltpu.sync_copy(acc.at[pl.ds(s * NZT, NZT)],
                        wex_o.at[pl.ds(s * NZT, NZT)])


def _sc_convscatter(num, wex, dst_r, zeros_f):
    mesh = plsc.VectorSubcoreMesh(core_axis_name="c", subcore_axis_name="s")
    return pl.kernel(
        _sc_convscatter_body,
        out_type=[jax.ShapeDtypeStruct((NPAD, F), jnp.float32),
                  jax.ShapeDtypeStruct((NPAD, F), jnp.float32)],
        mesh=mesh,
        scratch_types=[
            pltpu.VMEM_SHARED((NPAD, F), jnp.float32),
            pltpu.VMEM((REW,), jnp.int32),
            pltpu.VMEM((RCH, F), jnp.float32),
        ],
    )(num, wex, dst_r, zeros_f)


def _node_phase_body(num, wex, res0, rbf, P, mexp, wr, wo, bo,
                     wbf0, bbf0, wbf1, bbf1, wd, bd,
                     wa00, ba00, wa01, ba01, wa10, ba10, wa11, ba11,
                     out_o):
    agg = num[...] / (wex[...] + 1e-16)
    gate = rbf[...] @ wr[...]
    t = (agg * gate) @ wo[...] + bo[...]

    Pv = P[...]
    cnt = jnp.sum(Pv, axis=0)
    denom = jnp.maximum(cnt, 1.0) * float(F)
    s_g = lax.dot_general(Pv, t, (((0,), (0,)), ((), ())))
    mean_g = jnp.sum(s_g, axis=1) / denom
    mean_n = Pv @ mean_g[:, None]
    xc = t - mean_n
    r = jnp.sum(xc * xc, axis=1, keepdims=True)
    v_g = lax.dot_general(Pv, r, (((0,), (0,)), ((), ())))
    var_g = v_g / denom[:, None]
    var_n = Pv @ var_g
    t = xc / jnp.sqrt(var_n + 1e-8)

    h = _silu(t @ wbf0[...] + bbf0[...])
    h = _silu(h @ wbf1[...] + bbf1[...])
    t = t + h
    t = _silu(t @ wd[...] + bd[...])
    t = t + res0[...]
    h = _silu(t @ wa00[...] + ba00[...])
    h = _silu(h @ wa01[...] + ba01[...])
    t = t + h
    h = _silu(t @ wa10[...] + ba10[...])
    h = _silu(h @ wa11[...] + ba11[...])
    out_o[...] = t + h


def _node_phase(num, wex, res0, node_rbf, P, mexp, p, li):
    cp = p['convs'][li]
    bf = p['bf_skip'][li]
    af = p['af_skip'][li]
    def b(shape):
        return pl.BlockSpec(shape, lambda: (0,) * len(shape))
    args = [
        num, wex, res0, node_rbf, P, mexp,
        cp['rbf'], cp['o']['W'], cp['o']['b'][None, :],
        bf[0]['W'], bf[0]['b'][None, :], bf[1]['W'], bf[1]['b'][None, :],
        p['dense_bf'][li]['W'], p['dense_bf'][li]['b'][None, :],
        af[0][0]['W'], af[0][0]['b'][None, :], af[0][1]['W'], af[0][1]['b'][None, :],
        af[1][0]['W'], af[1][0]['b'][None, :], af[1][1]['W'], af[1][1]['b'][None, :],
    ]
    in_specs = [b(tuple(a.shape)) for a in args]
    return pl.pallas_call(
        _node_phase_body,
        in_specs=in_specs,
        out_specs=b((N, F)),
        out_shape=jax.ShapeDtypeStruct((N, F), jnp.float32),
    )(*args)


def _readout_pre_body(x, rbf, wu, bu, wg, y0_o, y1_o):
    up = _silu(x[...] @ wu[...] + bu[...])
    gate = _silu(rbf[...] @ wg[...])
    y = up * gate
    y0_o[...] = y[:, :F]
    y1_o[...] = y[:, F:]


def _readout_pre(x, node_rbf, p):
    row = lambda i: (i, 0)
    full = lambda i: (0, 0)
    return pl.pallas_call(
        _readout_pre_body,
        grid=(NNB,),
        in_specs=[pl.BlockSpec((BN, F), row), pl.BlockSpec((BN, RBF), row),
                  pl.BlockSpec((F, 2 * F), full), pl.BlockSpec((1, 2 * F), full),
                  pl.BlockSpec((RBF, 2 * F), full)],
        out_specs=[pl.BlockSpec((BN, F), row), pl.BlockSpec((BN, F), row)],
        out_shape=[jax.ShapeDtypeStruct((N, F), jnp.float32),
                   jax.ShapeDtypeStruct((N, F), jnp.float32)],
    )(x, node_rbf, p['up']['W'], p['up']['b'][None, :],
      p['readout']['gate'])


def _sc_readout_body(y0_hbm, y1_hbm, src_r, dst_r, zeros, h0_o, h1_o,
                     acc, srcbuf, dstbuf, rows, sem):
    c = lax.axis_index("c")
    s = lax.axis_index("s")
    pltpu.sync_copy(src_r.at[s], srcbuf)
    pltpu.sync_copy(dst_r.at[s], dstbuf)
    pltpu.sync_copy(zeros.at[pl.ds(s * NZT, NZT)], acc.at[pl.ds(s * NZT, NZT)])
    plsc.subcore_barrier()

    def chunk(j, y_hbm):
        pltpu.async_copy(y_hbm.at[srcbuf.at[pl.ds(j * RCH, RCH)]], rows,
                         sem).wait()
        pltpu.sync_copy(rows, acc.at[dstbuf.at[pl.ds(j * RCH, RCH)]],
                        add=True)

    @pl.when(c == 0)
    def _():
        def body(j, carry):
            chunk(j, y0_hbm)
            return carry
        lax.fori_loop(0, RNCH, body, 0)

    @pl.when(c == 1)
    def _():
        def body(j, carry):
            chunk(j, y1_hbm)
            return carry
        lax.fori_loop(0, RNCH, body, 0)

    plsc.subcore_barrier()

    @pl.when(c == 0)
    def _():
        pltpu.sync_copy(acc.at[pl.ds(s * NZT, NZT)], h0_o.at[pl.ds(s * NZT, NZT)])

    @pl.when(c == 1)
    def _():
        pltpu.sync_copy(acc.at[pl.ds(s * NZT, NZT)], h1_o.at[pl.ds(s * NZT, NZT)])


def _sc_readout(y0, y1, src_r, dst_r, zeros_f):
    mesh = plsc.VectorSubcoreMesh(core_axis_name="c", subcore_axis_name="s")
    return pl.kernel(
        _sc_readout_body,
        out_type=[jax.ShapeDtypeStruct((NPAD, F), jnp.float32),
                  jax.ShapeDtypeStruct((NPAD, F), jnp.float32)],
        mesh=mesh,
        scratch_types=[
            pltpu.VMEM_SHARED((NPAD, F), jnp.float32),
            pltpu.VMEM((REW,), jnp.int32),
            pltpu.VMEM((REW,), jnp.int32),
            pltpu.VMEM((RCH, F), jnp.float32),
            pltpu.SemaphoreType.DMA,
        ],
    )(y0, y1, src_r, dst_r, zeros_f)


def _readout_post_body(h0, h1, P0, w0, b0, w1, b1, w2, b2, wo, bo, out_o):
    t = jnp.concatenate([h0[...], h1[...]], axis=1)
    t = _silu(t @ w0[...] + b0[...])
    t = _silu(t @ w1[...] + b1[...])
    t = _silu(t @ w2[...] + b2[...])
    Pv = P0[...]
    cnt = jnp.sum(Pv, axis=0)
    pooled = lax.dot_general(Pv, t, (((0,), (0,)), ((), ())))
    pooled = pooled / jnp.maximum(cnt, 1.0)[:, None]
    out_o[...] = pooled @ wo[...] + bo[...]


def _readout_post(h0, h1, P0, p):
    rp = p['readout']
    def b(shape):
        return pl.BlockSpec(shape, lambda: (0,) * len(shape))
    args = [h0, h1, P0,
            rp['mlp'][0]['W'], rp['mlp'][0]['b'][None, :],
            rp['mlp'][1]['W'], rp['mlp'][1]['b'][None, :],
            rp['mlp'][2]['W'], rp['mlp'][2]['b'][None, :],
            rp['out']['W'], rp['out']['b'][None, :]]
    return pl.pallas_call(
        _readout_post_body,
        in_specs=[b(tuple(a.shape)) for a in args],
        out_specs=b((G, 1)),
        out_shape=jax.ShapeDtypeStruct((G, 1), jnp.float32),
    )(*args)


def kernel(x, edge_attr, edge_sbf, node_rbf, edge_index, batch, edge_index_0,
           atom_batch, params):
    p = params
    src_r = edge_index[0].reshape(NW, CEW)
    dst_r = edge_index[1].reshape(NW, CEW)
    dst_rs = edge_index[1].reshape(NS, REW)
    src0_r = edge_index_0[0].reshape(NS, REW)
    dst0_s = edge_index_0[1].reshape(NS, REW)

    mred = np.zeros((F, H), np.float32)
    for h in range(H):
        mred[h * DH:(h + 1) * DH, h] = 1.0
    mred = jnp.asarray(mred)
    mexp = jnp.asarray(mred.T)

    P = jax.nn.one_hot(batch, G, dtype=jnp.float32)
    P0 = jax.nn.one_hot(atom_batch, G, dtype=jnp.float32)
    zeros_f = jnp.zeros((NPAD, F), jnp.float32)

    ee0, ee1, sl0, sl1 = _edge_pre(edge_attr, edge_sbf, p)
    ees = [ee0, ee1]
    sls = [sl0, sl1]

    out = x
    for li in range(L):
        q, kv = _qkv(out, p['convs'][li])
        qd, kvs = _sc_convgather(q, kv, src_r, dst_r)
        num, wex = _edge_msg(qd, kvs, ees[li], sls[li], mred, mexp)
        num_a, wex_a = _sc_convscatter(num, wex, dst_rs, zeros_f)
        out = _node_phase(num_a[:N], wex_a[:N], out, node_rbf, P, mexp, p, li)

    y0, y1 = _readout_pre(out, node_rbf, p)
    h0, h1 = _sc_readout(y0, y1, src0_r, dst0_s, zeros_f)
    res = _readout_post(h0[:N], h1[:N], P0, p)
    return res.reshape(-1)


# speedup vs baseline: 5.2742x; 1.0173x over previous
import functools

import jax
import jax.numpy as jnp
import numpy as np
from jax import lax
from jax.experimental import pallas as pl
from jax.experimental.pallas import tpu as pltpu
from jax.experimental.pallas import tpu_sc as plsc

N = 10000
E = 160000
F = 128
EMB = 128
RBF = 16
SBF = 128
H = 8
DH = 16
L = 2
G = 64

BE = 2000
NEB = E // BE
BN = 2000
NNB = N // BN

NC = 2
NS = 16
NW = NC * NS
NPAD = 10240
NZT = NPAD // NS

ND = F + 2 * H

CEW = E // NW
CCH = 200
CNCH = CEW // CCH

REW = E // NS
RCH = 200
RNCH = REW // RCH


def _silu(x):
    return x * jax.nn.sigmoid(x)


def _edge_pre_body(attr, sbf, w0, b0, w1, b1, ws0, ws1,
                   ea_o, sl0_o, sl1_o):
    h = _silu(attr[...] @ w0[...] + b0[...])
    ea_o[...] = h @ w1[...] + b1[...]
    sl0_o[...] = sbf[...] @ ws0[...]
    sl1_o[...] = sbf[...] @ ws1[...]


def _edge_pre(edge_attr, edge_sbf, p):
    row = lambda i: (i, 0)
    full = lambda i: (0, 0)
    eb = pl.BlockSpec((BE, F), row)
    wb = pl.BlockSpec((F, F), full)
    bb = pl.BlockSpec((1, F), full)
    sb = pl.BlockSpec((F, H), full)
    ob8 = pl.BlockSpec((BE, H), row)
    return pl.pallas_call(
        _edge_pre_body,
        grid=(NEB,),
        in_specs=[eb, eb, wb, bb, wb, bb, sb, sb],
        out_specs=[eb, ob8, ob8],
        out_shape=[
            jax.ShapeDtypeStruct((E, F), jnp.float32),
            jax.ShapeDtypeStruct((E, H), jnp.float32),
            jax.ShapeDtypeStruct((E, H), jnp.float32),
        ],
    )(edge_attr, edge_sbf,
      p['edgenn'][0]['W'], p['edgenn'][0]['b'][None, :],
      p['edgenn'][1]['W'], p['edgenn'][1]['b'][None, :],
      p['convs'][0]['sbf'], p['convs'][1]['sbf'])


def _qkv_body(x, wq, bq, wk, bk, wv, bv, q_o, kv_o):
    xv = x[...]
    q_o[...] = xv @ wq[...] + bq[...]
    k = xv @ wk[...] + bk[...]
    v = xv @ wv[...] + bv[...]
    kv_o[...] = jnp.concatenate([k, v], axis=1)


def _qkv(x, cp):
    row = lambda i: (i, 0)
    full = lambda i: (0, 0)
    nb = pl.BlockSpec((BN, F), row)
    wb = pl.BlockSpec((F, F), full)
    bb = pl.BlockSpec((1, F), full)
    return pl.pallas_call(
        _qkv_body,
        grid=(NNB,),
        in_specs=[nb, wb, bb, wb, bb, wb, bb],
        out_specs=[nb, pl.BlockSpec((BN, 2 * F), row)],
        out_shape=[jax.ShapeDtypeStruct((N, F), jnp.float32),
                   jax.ShapeDtypeStruct((N, 2 * F), jnp.float32)],
    )(x, cp['q']['W'], cp['q']['b'][None, :],
      cp['k']['W'], cp['k']['b'][None, :],
      cp['v']['W'], cp['v']['b'][None, :])


def _sc_convgather_body(q_hbm, kv_hbm, src_r, dst_r, qd_o, kvs_o,
                        srcbuf, dstbuf, qbuf, kvbuf, sem, sem2):
    c = lax.axis_index("c")
    s = lax.axis_index("s")
    w = c * NS + s
    pltpu.sync_copy(src_r.at[w], srcbuf)
    pltpu.sync_copy(dst_r.at[w], dstbuf)

    def body(j, carry):
        base = w * CEW + j * CCH
        cp1 = pltpu.async_copy(q_hbm.at[dstbuf.at[pl.ds(j * CCH, CCH)]], qbuf, sem)
        cp2 = pltpu.async_copy(kv_hbm.at[srcbuf.at[pl.ds(j * CCH, CCH)]], kvbuf, sem2)
        cp1.wait()
        cp2.wait()
        pltpu.sync_copy(qbuf, qd_o.at[pl.ds(base, CCH)])
        pltpu.sync_copy(kvbuf, kvs_o.at[pl.ds(base, CCH)])
        return carry

    lax.fori_loop(0, CNCH, body, 0)


def _sc_convgather(q, kv, src_r, dst_r):
    mesh = plsc.VectorSubcoreMesh(core_axis_name="c", subcore_axis_name="s")
    return pl.kernel(
        _sc_convgather_body,
        out_type=[jax.ShapeDtypeStruct((E, F), jnp.float32),
                  jax.ShapeDtypeStruct((E, 2 * F), jnp.float32)],
        mesh=mesh,
        scratch_types=[
            pltpu.VMEM((CEW,), jnp.int32),
            pltpu.VMEM((CEW,), jnp.int32),
            pltpu.VMEM((CCH, F), jnp.float32),
            pltpu.VMEM((CCH, 2 * F), jnp.float32),
            pltpu.SemaphoreType.DMA,
            pltpu.SemaphoreType.DMA,
        ],
    )(q, kv, src_r, dst_r)


def _edge_msg_body(qd, kvs, ea, sl, we, be, mred, mexp, num_o, wex_o):
    eev = ea[...] @ we[...] + be[...]
    kvv = kvs[...]
    k = kvv[:, :F] + eev
    v = kvv[:, F:] + eev
    logit = (qd[...] * k) @ mred[...] * (1.0 / np.sqrt(float(DH))) + sl[...]
    wex = jnp.exp(logit) @ mexp[...]
    wex_o[...] = wex
    num_o[...] = wex * v


def _edge_msg(qd, kvs, ea, sl, cp, mred, mexp):
    row = lambda i: (i, 0)
    full = lambda i: (0, 0)
    return pl.pallas_call(
        _edge_msg_body,
        grid=(NEB,),
        in_specs=[pl.BlockSpec((BE, F), row), pl.BlockSpec((BE, 2 * F), row),
                  pl.BlockSpec((BE, F), row), pl.BlockSpec((BE, H), row),
                  pl.BlockSpec((F, F), full), pl.BlockSpec((1, F), full),
                  pl.BlockSpec((F, H), full), pl.BlockSpec((H, F), full)],
        out_specs=[pl.BlockSpec((BE, F), row), pl.BlockSpec((BE, F), row)],
        out_shape=[jax.ShapeDtypeStruct((E, F), jnp.float32),
                   jax.ShapeDtypeStruct((E, F), jnp.float32)],
    )(qd, kvs, ea, sl, cp['e']['W'], cp['e']['b'][None, :], mred, mexp)


def _sc_convscatter_body(num_hbm, wex_hbm, dst_r, zeros, num_o, wex_o,
                         acc, dstbuf, rowbuf):
    c = lax.axis_index("c")
    s = lax.axis_index("s")
    pltpu.sync_copy(dst_r.at[s], dstbuf)
    pltpu.sync_copy(zeros.at[pl.ds(s * NZT, NZT)], acc.at[pl.ds(s * NZT, NZT)])
    plsc.subcore_barrier()

    def chunk(j, src_hbm):
        base = s * REW + j * RCH
        pltpu.sync_copy(src_hbm.at[pl.ds(base, RCH)], rowbuf)
        pltpu.sync_copy(rowbuf, acc.at[dstbuf.at[pl.ds(j * RCH, RCH)]],
                        add=True)

    @pl.when(c == 0)
    def _():
        def body(j, carry):
            chunk(j, num_hbm)
            return carry
        lax.fori_loop(0, RNCH, body, 0)

    @pl.when(c == 1)
    def _():
        def body(j, carry):
            chunk(j, wex_hbm)
            return carry
        lax.fori_loop(0, RNCH, body, 0)

    plsc.subcore_barrier()

    @pl.when(c == 0)
    def _():
        pltpu.sync_copy(acc.at[pl.ds(s * NZT, NZT)],
                        num_o.at[pl.ds(s * NZT, NZT)])

    @pl.when(c == 1)
    def _():
        pltpu.sync_copy(acc.at[pl.ds(s * NZT, NZT)],
                        wex_o.at[pl.ds(s * NZT, NZT)])


def _sc_convscatter(num, wex, dst_r, zeros_f):
    mesh = plsc.VectorSubcoreMesh(core_axis_name="c", subcore_axis_name="s")
    return pl.kernel(
        _sc_convscatter_body,
        out_type=[jax.ShapeDtypeStruct((NPAD, F), jnp.float32),
                  jax.ShapeDtypeStruct((NPAD, F), jnp.float32)],
        mesh=mesh,
        scratch_types=[
            pltpu.VMEM_SHARED((NPAD, F), jnp.float32),
            pltpu.VMEM((REW,), jnp.int32),
            pltpu.VMEM((RCH, F), jnp.float32),
        ],
    )(num, wex, dst_r, zeros_f)


def _node_phase_body(num, wex, res0, rbf, P, mexp, wr, wo, bo,
                     wbf0, bbf0, wbf1, bbf1, wd, bd,
                     wa00, ba00, wa01, ba01, wa10, ba10, wa11, ba11,
                     out_o):
    agg = num[...] / (wex[...] + 1e-16)
    gate = rbf[...] @ wr[...]
    t = (agg * gate) @ wo[...] + bo[...]

    Pv = P[...]
    cnt = jnp.sum(Pv, axis=0)
    denom = jnp.maximum(cnt, 1.0) * float(F)
    s_g = lax.dot_general(Pv, t, (((0,), (0,)), ((), ())))
    mean_g = jnp.sum(s_g, axis=1) / denom
    mean_n = Pv @ mean_g[:, None]
    xc = t - mean_n
    r = jnp.sum(xc * xc, axis=1, keepdims=True)
    v_g = lax.dot_general(Pv, r, (((0,), (0,)), ((), ())))
    var_g = v_g / denom[:, None]
    var_n = Pv @ var_g
    t = xc / jnp.sqrt(var_n + 1e-8)

    h = _silu(t @ wbf0[...] + bbf0[...])
    h = _silu(h @ wbf1[...] + bbf1[...])
    t = t + h
    t = _silu(t @ wd[...] + bd[...])
    t = t + res0[...]
    h = _silu(t @ wa00[...] + ba00[...])
    h = _silu(h @ wa01[...] + ba01[...])
    t = t + h
    h = _silu(t @ wa10[...] + ba10[...])
    h = _silu(h @ wa11[...] + ba11[...])
    out_o[...] = t + h


def _node_phase(num, wex, res0, node_rbf, P, mexp, p, li):
    cp = p['convs'][li]
    bf = p['bf_skip'][li]
    af = p['af_skip'][li]
    def b(shape):
        return pl.BlockSpec(shape, lambda: (0,) * len(shape))
    args = [
        num, wex, res0, node_rbf, P, mexp,
        cp['rbf'], cp['o']['W'], cp['o']['b'][None, :],
        bf[0]['W'], bf[0]['b'][None, :], bf[1]['W'], bf[1]['b'][None, :],
        p['dense_bf'][li]['W'], p['dense_bf'][li]['b'][None, :],
        af[0][0]['W'], af[0][0]['b'][None, :], af[0][1]['W'], af[0][1]['b'][None, :],
        af[1][0]['W'], af[1][0]['b'][None, :], af[1][1]['W'], af[1][1]['b'][None, :],
    ]
    in_specs = [b(tuple(a.shape)) for a in args]
    return pl.pallas_call(
        _node_phase_body,
        in_specs=in_specs,
        out_specs=b((N, F)),
        out_shape=jax.ShapeDtypeStruct((N, F), jnp.float32),
    )(*args)


def _readout_pre_body(x, rbf, wu, bu, wg, y0_o, y1_o):
    up = _silu(x[...] @ wu[...] + bu[...])
    gate = _silu(rbf[...] @ wg[...])
    y = up * gate
    y0_o[...] = y[:, :F]
    y1_o[...] = y[:, F:]


def _readout_pre(x, node_rbf, p):
    row = lambda i: (i, 0)
    full = lambda i: (0, 0)
    return pl.pallas_call(
        _readout_pre_body,
        grid=(NNB,),
        in_specs=[pl.BlockSpec((BN, F), row), pl.BlockSpec((BN, RBF), row),
                  pl.BlockSpec((F, 2 * F), full), pl.BlockSpec((1, 2 * F), full),
                  pl.BlockSpec((RBF, 2 * F), full)],
        out_specs=[pl.BlockSpec((BN, F), row), pl.BlockSpec((BN, F), row)],
        out_shape=[jax.ShapeDtypeStruct((N, F), jnp.float32),
                   jax.ShapeDtypeStruct((N, F), jnp.float32)],
    )(x, node_rbf, p['up']['W'], p['up']['b'][None, :],
      p['readout']['gate'])


def _sc_readout_body(y0_hbm, y1_hbm, src_r, dst_r, zeros, h0_o, h1_o,
                     acc, srcbuf, dstbuf, rows, sem):
    c = lax.axis_index("c")
    s = lax.axis_index("s")
    pltpu.sync_copy(src_r.at[s], srcbuf)
    pltpu.sync_copy(dst_r.at[s], dstbuf)
    pltpu.sync_copy(zeros.at[pl.ds(s * NZT, NZT)], acc.at[pl.ds(s * NZT, NZT)])
    plsc.subcore_barrier()

    def chunk(j, y_hbm):
        pltpu.async_copy(y_hbm.at[srcbuf.at[pl.ds(j * RCH, RCH)]], rows,
                         sem).wait()
        pltpu.sync_copy(rows, acc.at[dstbuf.at[pl.ds(j * RCH, RCH)]],
                        add=True)

    @pl.when(c == 0)
    def _():
        def body(j, carry):
            chunk(j, y0_hbm)
            return carry
        lax.fori_loop(0, RNCH, body, 0)

    @pl.when(c == 1)
    def _():
        def body(j, carry):
            chunk(j, y1_hbm)
            return carry
        lax.fori_loop(0, RNCH, body, 0)

    plsc.subcore_barrier()

    @pl.when(c == 0)
    def _():
        pltpu.sync_copy(acc.at[pl.ds(s * NZT, NZT)], h0_o.at[pl.ds(s * NZT, NZT)])

    @pl.when(c == 1)
    def _():
        pltpu.sync_copy(acc.at[pl.ds(s * NZT, NZT)], h1_o.at[pl.ds(s * NZT, NZT)])


def _sc_readout(y0, y1, src_r, dst_r, zeros_f):
    mesh = plsc.VectorSubcoreMesh(core_axis_name="c", subcore_axis_name="s")
    return pl.kernel(
        _sc_readout_body,
        out_type=[jax.ShapeDtypeStruct((NPAD, F), jnp.float32),
                  jax.ShapeDtypeStruct((NPAD, F), jnp.float32)],
        mesh=mesh,
        scratch_types=[
            pltpu.VMEM_SHARED((NPAD, F), jnp.float32),
            pltpu.VMEM((REW,), jnp.int32),
            pltpu.VMEM((REW,), jnp.int32),
            pltpu.VMEM((RCH, F), jnp.float32),
            pltpu.SemaphoreType.DMA,
        ],
    )(y0, y1, src_r, dst_r, zeros_f)


def _readout_post_body(h0, h1, P0, w0, b0, w1, b1, w2, b2, wo, bo, out_o):
    t = jnp.concatenate([h0[...], h1[...]], axis=1)
    t = _silu(t @ w0[...] + b0[...])
    t = _silu(t @ w1[...] + b1[...])
    t = _silu(t @ w2[...] + b2[...])
    Pv = P0[...]
    cnt = jnp.sum(Pv, axis=0)
    pooled = lax.dot_general(Pv, t, (((0,), (0,)), ((), ())))
    pooled = pooled / jnp.maximum(cnt, 1.0)[:, None]
    out_o[...] = pooled @ wo[...] + bo[...]


def _readout_post(h0, h1, P0, p):
    rp = p['readout']
    def b(shape):
        return pl.BlockSpec(shape, lambda: (0,) * len(shape))
    args = [h0, h1, P0,
            rp['mlp'][0]['W'], rp['mlp'][0]['b'][None, :],
            rp['mlp'][1]['W'], rp['mlp'][1]['b'][None, :],
            rp['mlp'][2]['W'], rp['mlp'][2]['b'][None, :],
            rp['out']['W'], rp['out']['b'][None, :]]
    return pl.pallas_call(
        _readout_post_body,
        in_specs=[b(tuple(a.shape)) for a in args],
        out_specs=b((G, 1)),
        out_shape=jax.ShapeDtypeStruct((G, 1), jnp.float32),
    )(*args)


def kernel(x, edge_attr, edge_sbf, node_rbf, edge_index, batch, edge_index_0,
           atom_batch, params):
    p = params
    src_r = edge_index[0].reshape(NW, CEW)
    dst_r = edge_index[1].reshape(NW, CEW)
    dst_rs = edge_index[1].reshape(NS, REW)
    src0_r = edge_index_0[0].reshape(NS, REW)
    dst0_s = edge_index_0[1].reshape(NS, REW)

    mred = np.zeros((F, H), np.float32)
    for h in range(H):
        mred[h * DH:(h + 1) * DH, h] = 1.0
    mred = jnp.asarray(mred)
    mexp = jnp.asarray(mred.T)

    P = jax.nn.one_hot(batch, G, dtype=jnp.float32)
    P0 = jax.nn.one_hot(atom_batch, G, dtype=jnp.float32)
    zeros_f = jnp.zeros((NPAD, F), jnp.float32)

    ea, sl0, sl1 = _edge_pre(edge_attr, edge_sbf, p)
    sls = [sl0, sl1]

    out = x
    for li in range(L):
        q, kv = _qkv(out, p['convs'][li])
        qd, kvs = _sc_convgather(q, kv, src_r, dst_r)
        num, wex = _edge_msg(qd, kvs, ea, sls[li], p['convs'][li], mred, mexp)
        num_a, wex_a = _sc_convscatter(num, wex, dst_rs, zeros_f)
        out = _node_phase(num_a[:N], wex_a[:N], out, node_rbf, P, mexp, p, li)

    y0, y1 = _readout_pre(out, node_rbf, p)
    h0, h1 = _sc_readout(y0, y1, src0_r, dst0_s, zeros_f)
    res = _readout_post(h0[:N], h1[:N], P0, p)
    return res.reshape(-1)
